# baseline (speedup 1.0000x reference)
import functools
import math

import jax
import jax.numpy as jnp
from jax import lax
from jax.experimental import pallas as pl
from jax.experimental.pallas import tpu as pltpu
from jax.experimental.pallas import tpu_sc as plsc

N = 10000
E = 320000
D = 128
DE = 16
INV_SQRT_DEG = 1.0 / math.sqrt(32.0)
LOG2 = math.log(2.0)

NC = 2
NS = 16
NW = NC * NS
EPW = E // NW
CHUNK = 80
NCHUNK = EPW // CHUNK
ROWS_PER_TILE = N // NS
ZROWS = 25


def _mm_kernel(x_ref, w_ref, o_ref):
    o_ref[...] = jnp.dot(x_ref[...], w_ref[...],
                         preferred_element_type=jnp.float32)


def _pack_bf16_pairs(z):
    u = lax.bitcast_convert_type(z.astype(jnp.bfloat16),
                                 jnp.uint16).astype(jnp.uint32)
    u3 = u.reshape(u.shape[0] // 2, 2, D)
    return (u3[:, 0, :] | (u3[:, 1, :] << 16)).astype(jnp.int32)


def _ew_kernel(ea_ref, w_ref, o_ref):
    e2 = ea_ref[...]
    ea = e2.reshape(2 * e2.shape[0], DE)
    o_ref[...] = _pack_bf16_pairs(
        jnp.dot(ea, w_ref[...], preferred_element_type=jnp.float32))


def _ew_packed(edge_attr, we, block_rows=2000):
    ea3 = edge_attr.reshape(E // 2, 2, DE)
    return pl.pallas_call(
        _ew_kernel,
        grid=(E // 2 // block_rows,),
        in_specs=[
            pl.BlockSpec((block_rows, 2, DE), lambda i: (i, 0, 0)),
            pl.BlockSpec((DE, D), lambda i: (0, 0)),
        ],
        out_specs=pl.BlockSpec((block_rows, D), lambda i: (i, 0)),
        out_shape=jax.ShapeDtypeStruct((E // 2, D), jnp.int32),
    )(ea3, we)


def _matmul(x, w, block_rows):
    rows, k = x.shape
    kk, cols = w.shape
    return pl.pallas_call(
        _mm_kernel,
        grid=(rows // block_rows,),
        in_specs=[
            pl.BlockSpec((block_rows, k), lambda i: (i, 0)),
            pl.BlockSpec((kk, cols), lambda i: (0, 0)),
        ],
        out_specs=pl.BlockSpec((block_rows, cols), lambda i: (i, 0)),
        out_shape=jax.ShapeDtypeStruct((rows, cols), jnp.float32),
    )(x, w)


def _epilogue_kernel(p_ref, w_ref, x_ref, o_ref):
    a = (p_ref[0] + p_ref[1]) * INV_SQRT_DEG
    z = jnp.dot(a, w_ref[...], preferred_element_type=jnp.float32)
    o_ref[...] = x_ref[...] + jax.nn.softplus(z) - LOG2


def _epilogue(partials, w2, x, block_rows=1000):
    return pl.pallas_call(
        _epilogue_kernel,
        grid=(N // block_rows,),
        in_specs=[
            pl.BlockSpec((NC, block_rows, D), lambda i: (0, i, 0)),
            pl.BlockSpec((D, D), lambda i: (0, 0)),
            pl.BlockSpec((block_rows, D), lambda i: (i, 0)),
        ],
        out_specs=pl.BlockSpec((block_rows, D), lambda i: (i, 0)),
        out_shape=jax.ShapeDtypeStruct((N, D), jnp.float32),
    )(partials, w2, x)


def _epi_h_kernel(p_ref, w_ref, x_ref, w1_ref, x_out, h_out):
    a = (p_ref[0] + p_ref[1]) * INV_SQRT_DEG
    z = jnp.dot(a, w_ref[...], preferred_element_type=jnp.float32)
    xn = x_ref[...] + jax.nn.softplus(z) - LOG2
    x_out[...] = xn
    h_out[...] = jnp.dot(xn, w1_ref[...], preferred_element_type=jnp.float32)


def _epilogue_h(partials, w2, x, w1_next, block_rows=1000):
    return pl.pallas_call(
        _epi_h_kernel,
        grid=(N // block_rows,),
        in_specs=[
            pl.BlockSpec((NC, block_rows, D), lambda i: (0, i, 0)),
            pl.BlockSpec((D, D), lambda i: (0, 0)),
            pl.BlockSpec((block_rows, D), lambda i: (i, 0)),
            pl.BlockSpec((D, D), lambda i: (0, 0)),
        ],
        out_specs=[
            pl.BlockSpec((block_rows, D), lambda i: (i, 0)),
            pl.BlockSpec((block_rows, D), lambda i: (i, 0)),
        ],
        out_shape=[
            jax.ShapeDtypeStruct((N, D), jnp.float32),
            jax.ShapeDtypeStruct((N, D), jnp.float32),
        ],
    )(partials, w2, x, w1_next)


NBUF = 3
NEB = 2


def _sc_body(h_hbm, ew_hbm, pk_hbm, out_hbm,
             sv, dv, pkv, hb, eb, zero_v, agg_sh, sg, se, ss, sp):
    cid = lax.axis_index("c")
    sid = lax.axis_index("s")
    wid = sid * NC + cid

    zline = jnp.zeros((16,), jnp.float32)

    def _zrow(r, _):
        for c in range(D // 16):
            zero_v[r, pl.ds(c * 16, 16)] = zline
        return 0

    lax.fori_loop(0, ZROWS, _zrow, 0)

    def _zfire(b, _):
        pltpu.async_copy(
            zero_v,
            agg_sh.at[pl.ds(sid * ROWS_PER_TILE + b * ZROWS, ZROWS), :], sg)
        return 0

    def _zdrain(b, _):
        pltpu.make_async_copy(
            zero_v, agg_sh.at[pl.ds(sid * ROWS_PER_TILE, ZROWS), :], sg).wait()
        return 0

    lax.fori_loop(0, ROWS_PER_TILE // ZROWS, _zfire, 0)
    lax.fori_loop(0, ROWS_PER_TILE // ZROWS, _zdrain, 0)
    pltpu.async_copy(
        pk_hbm.at[pl.ds(wid * NCHUNK * CHUNK, CHUNK)], pkv.at[0], sp)
    plsc.subcore_barrier()

    def _iter(j, _):
        b = j % NBUF
        be = j % NEB

        @pl.when(j >= NBUF)
        def _():
            pltpu.make_async_copy(hb.at[b], agg_sh.at[dv.at[b]], ss).wait()

        @pl.when(j < NCHUNK)
        def _():
            bp = j % 2
            pltpu.make_async_copy(
                pk_hbm.at[pl.ds(0, CHUNK)], pkv.at[bp], sp).wait()

            @pl.when(j + 1 < NCHUNK)
            def _():
                pltpu.async_copy(
                    pk_hbm.at[pl.ds((wid * NCHUNK + j + 1) * CHUNK, CHUNK)],
                    pkv.at[1 - bp], sp)

            for i in range(CHUNK // 16):
                s = pl.ds(i * 16, 16)
                v = pkv[bp, s]
                sv[be, s] = v & 0x3FFF
                dv[b, s] = v >> 14
            pltpu.async_copy(h_hbm.at[sv.at[be]], hb.at[b], sg)
            pltpu.async_copy(
                ew_hbm.at[pl.ds((wid * NCHUNK + j) * (CHUNK // 2),
                                CHUNK // 2), :],
                eb.at[be], se)

        @pl.when(j >= 1)
        def _():
            b1 = (j - 1) % NBUF
            be1 = (j - 1) % NEB
            pltpu.make_async_copy(h_hbm.at[sv.at[be1]], hb.at[b1], sg).wait()
            pltpu.make_async_copy(
                ew_hbm.at[pl.ds(0, CHUNK // 2), :], eb.at[be1], se).wait()

            @plsc.parallel_loop(0, CHUNK // 2, step=1, unroll=2)
            def _mul(p):
                w = [eb[be1, p, pl.ds(c * 16, 16)] for c in range(D // 16)]
                h0 = [hb[b1, 2 * p, pl.ds(c * 16, 16)]
                      for c in range(D // 16)]
                h1 = [hb[b1, 2 * p + 1, pl.ds(c * 16, 16)]
                      for c in range(D // 16)]
                for c in range(D // 16):
                    lo = lax.bitcast_convert_type(w[c] << 16, jnp.float32)
                    hi = lax.bitcast_convert_type(
                        w[c] & jnp.int32(-65536), jnp.float32)
                    hb[b1, 2 * p, pl.ds(c * 16, 16)] = h0[c] * lo
                    hb[b1, 2 * p + 1, pl.ds(c * 16, 16)] = h1[c] * hi
            pltpu.async_copy(hb.at[b1], agg_sh.at[dv.at[b1]], ss, add=True)

        return 0

    lax.fori_loop(0, NCHUNK + 1, _iter, 0)
    pltpu.make_async_copy(hb.at[0], agg_sh.at[dv.at[0]], ss).wait()
    pltpu.make_async_copy(hb.at[0], agg_sh.at[dv.at[0]], ss).wait()
    plsc.subcore_barrier()

    pltpu.sync_copy(
        agg_sh.at[pl.ds(sid * ROWS_PER_TILE, ROWS_PER_TILE), :],
        out_hbm.at[cid, sid])


_sc_segment = functools.partial(
    pl.kernel,
    out_type=jax.ShapeDtypeStruct((NC, NS, ROWS_PER_TILE, D), jnp.float32),
    mesh=plsc.VectorSubcoreMesh(core_axis_name="c", subcore_axis_name="s"),
    scratch_types=[
        pltpu.VMEM((NEB, CHUNK), jnp.int32),
        pltpu.VMEM((NBUF, CHUNK), jnp.int32),
        pltpu.VMEM((2, CHUNK), jnp.int32),
        pltpu.VMEM((NBUF, CHUNK, D), jnp.float32),
        pltpu.VMEM((NEB, CHUNK // 2, D), jnp.int32),
        pltpu.VMEM((ZROWS, D), jnp.float32),
        pltpu.VMEM_SHARED((N, D), jnp.float32),
        pltpu.SemaphoreType.DMA,
        pltpu.SemaphoreType.DMA,
        pltpu.SemaphoreType.DMA,
        pltpu.SemaphoreType.DMA,
    ],
)(_sc_body)


def kernel(x, edge_attr, edge_index, W1_0, We_0, W2_0, W1_1, We_1, W2_1):
    packed = (edge_index[1] << 14) | edge_index[0]

    ew0 = _ew_packed(edge_attr, We_0)
    h0 = _matmul(x, W1_0, block_rows=1000)
    p0 = _sc_segment(h0, ew0, packed).reshape(NC, N, D)
    ew1 = _ew_packed(edge_attr, We_1)
    x1, h1 = _epilogue_h(p0, W2_0, x, W1_1)
    p1 = _sc_segment(h1, ew1, packed).reshape(NC, N, D)
    return _epilogue(p1, W2_1, x1)


# speedup vs baseline: 1.0095x; 1.0095x over previous
import functools
import math

import jax
import jax.numpy as jnp
from jax import lax
from jax.experimental import pallas as pl
from jax.experimental.pallas import tpu as pltpu
from jax.experimental.pallas import tpu_sc as plsc

N = 10000
E = 320000
D = 128
DE = 16
INV_SQRT_DEG = 1.0 / math.sqrt(32.0)
LOG2 = math.log(2.0)

NC = 2
NS = 16
NW = NC * NS
EPW = E // NW
CHUNK = 80
NCHUNK = EPW // CHUNK
ROWS_PER_TILE = N // NS
ZROWS = 25


def _mm_kernel(x_ref, w_ref, o_ref):
    o_ref[...] = jnp.dot(x_ref[...], w_ref[...],
                         preferred_element_type=jnp.float32)


def _pack_bf16_pairs(z):
    u = lax.bitcast_convert_type(z.astype(jnp.bfloat16),
                                 jnp.uint16).astype(jnp.uint32)
    u3 = u.reshape(u.shape[0] // 2, 2, D)
    return (u3[:, 0, :] | (u3[:, 1, :] << 16)).astype(jnp.int32)


def _ew_kernel(ea_ref, w_ref, o_ref):
    e2 = ea_ref[...]
    ea = e2.reshape(2 * e2.shape[0], DE)
    o_ref[...] = _pack_bf16_pairs(
        jnp.dot(ea, w_ref[...], preferred_element_type=jnp.float32))


def _ew_packed(edge_attr, we, block_rows=8000):
    ea3 = edge_attr.reshape(E // 2, 2, DE)
    return pl.pallas_call(
        _ew_kernel,
        grid=(E // 2 // block_rows,),
        in_specs=[
            pl.BlockSpec((block_rows, 2, DE), lambda i: (i, 0, 0)),
            pl.BlockSpec((DE, D), lambda i: (0, 0)),
        ],
        out_specs=pl.BlockSpec((block_rows, D), lambda i: (i, 0)),
        out_shape=jax.ShapeDtypeStruct((E // 2, D), jnp.int32),
    )(ea3, we)


def _matmul(x, w, block_rows):
    rows, k = x.shape
    kk, cols = w.shape
    return pl.pallas_call(
        _mm_kernel,
        grid=(rows // block_rows,),
        in_specs=[
            pl.BlockSpec((block_rows, k), lambda i: (i, 0)),
            pl.BlockSpec((kk, cols), lambda i: (0, 0)),
        ],
        out_specs=pl.BlockSpec((block_rows, cols), lambda i: (i, 0)),
        out_shape=jax.ShapeDtypeStruct((rows, cols), jnp.float32),
    )(x, w)


def _epilogue_kernel(p_ref, w_ref, x_ref, o_ref):
    a = (p_ref[0] + p_ref[1]) * INV_SQRT_DEG
    z = jnp.dot(a, w_ref[...], preferred_element_type=jnp.float32)
    o_ref[...] = x_ref[...] + jax.nn.softplus(z) - LOG2


def _epilogue(partials, w2, x, block_rows=1000):
    return pl.pallas_call(
        _epilogue_kernel,
        grid=(N // block_rows,),
        in_specs=[
            pl.BlockSpec((NC, block_rows, D), lambda i: (0, i, 0)),
            pl.BlockSpec((D, D), lambda i: (0, 0)),
            pl.BlockSpec((block_rows, D), lambda i: (i, 0)),
        ],
        out_specs=pl.BlockSpec((block_rows, D), lambda i: (i, 0)),
        out_shape=jax.ShapeDtypeStruct((N, D), jnp.float32),
    )(partials, w2, x)


def _epi_h_kernel(p_ref, w_ref, x_ref, w1_ref, x_out, h_out):
    a = (p_ref[0] + p_ref[1]) * INV_SQRT_DEG
    z = jnp.dot(a, w_ref[...], preferred_element_type=jnp.float32)
    xn = x_ref[...] + jax.nn.softplus(z) - LOG2
    x_out[...] = xn
    h_out[...] = jnp.dot(xn, w1_ref[...], preferred_element_type=jnp.float32)


def _epilogue_h(partials, w2, x, w1_next, block_rows=1000):
    return pl.pallas_call(
        _epi_h_kernel,
        grid=(N // block_rows,),
        in_specs=[
            pl.BlockSpec((NC, block_rows, D), lambda i: (0, i, 0)),
            pl.BlockSpec((D, D), lambda i: (0, 0)),
            pl.BlockSpec((block_rows, D), lambda i: (i, 0)),
            pl.BlockSpec((D, D), lambda i: (0, 0)),
        ],
        out_specs=[
            pl.BlockSpec((block_rows, D), lambda i: (i, 0)),
            pl.BlockSpec((block_rows, D), lambda i: (i, 0)),
        ],
        out_shape=[
            jax.ShapeDtypeStruct((N, D), jnp.float32),
            jax.ShapeDtypeStruct((N, D), jnp.float32),
        ],
    )(partials, w2, x, w1_next)


NBUF = 3
NEB = 2


def _sc_body(h_hbm, ew_hbm, pk_hbm, out_hbm,
             sv, dv, pkv, hb, eb, zero_v, agg_sh, sg, se, ss, sp):
    cid = lax.axis_index("c")
    sid = lax.axis_index("s")
    wid = sid * NC + cid

    zline = jnp.zeros((16,), jnp.float32)

    def _zrow(r, _):
        for c in range(D // 16):
            zero_v[r, pl.ds(c * 16, 16)] = zline
        return 0

    lax.fori_loop(0, ZROWS, _zrow, 0)

    def _zfire(b, _):
        pltpu.async_copy(
            zero_v,
            agg_sh.at[pl.ds(sid * ROWS_PER_TILE + b * ZROWS, ZROWS), :], sg)
        return 0

    def _zdrain(b, _):
        pltpu.make_async_copy(
            zero_v, agg_sh.at[pl.ds(sid * ROWS_PER_TILE, ZROWS), :], sg).wait()
        return 0

    lax.fori_loop(0, ROWS_PER_TILE // ZROWS, _zfire, 0)
    lax.fori_loop(0, ROWS_PER_TILE // ZROWS, _zdrain, 0)
    pltpu.async_copy(
        pk_hbm.at[pl.ds(wid * NCHUNK * CHUNK, CHUNK)], pkv.at[0], sp)
    plsc.subcore_barrier()

    def _iter(j, _):
        b = j % NBUF
        be = j % NEB

        @pl.when(j >= NBUF)
        def _():
            pltpu.make_async_copy(hb.at[b], agg_sh.at[dv.at[b]], ss).wait()

        @pl.when(j < NCHUNK)
        def _():
            bp = j % 2
            pltpu.make_async_copy(
                pk_hbm.at[pl.ds(0, CHUNK)], pkv.at[bp], sp).wait()

            @pl.when(j + 1 < NCHUNK)
            def _():
                pltpu.async_copy(
                    pk_hbm.at[pl.ds((wid * NCHUNK + j + 1) * CHUNK, CHUNK)],
                    pkv.at[1 - bp], sp)

            for i in range(CHUNK // 16):
                s = pl.ds(i * 16, 16)
                v = pkv[bp, s]
                sv[be, s] = v & 0x3FFF
                dv[b, s] = v >> 14
            pltpu.async_copy(h_hbm.at[sv.at[be]], hb.at[b], sg)
            pltpu.async_copy(
                ew_hbm.at[pl.ds((wid * NCHUNK + j) * (CHUNK // 2),
                                CHUNK // 2), :],
                eb.at[be], se)

        @pl.when(j >= 1)
        def _():
            b1 = (j - 1) % NBUF
            be1 = (j - 1) % NEB
            pltpu.make_async_copy(h_hbm.at[sv.at[be1]], hb.at[b1], sg).wait()
            pltpu.make_async_copy(
                ew_hbm.at[pl.ds(0, CHUNK // 2), :], eb.at[be1], se).wait()

            @plsc.parallel_loop(0, CHUNK // 2, step=1, unroll=2)
            def _mul(p):
                w = [eb[be1, p, pl.ds(c * 16, 16)] for c in range(D // 16)]
                h0 = [hb[b1, 2 * p, pl.ds(c * 16, 16)]
                      for c in range(D // 16)]
                h1 = [hb[b1, 2 * p + 1, pl.ds(c * 16, 16)]
                      for c in range(D // 16)]
                for c in range(D // 16):
                    lo = lax.bitcast_convert_type(w[c] << 16, jnp.float32)
                    hi = lax.bitcast_convert_type(
                        w[c] & jnp.int32(-65536), jnp.float32)
                    hb[b1, 2 * p, pl.ds(c * 16, 16)] = h0[c] * lo
                    hb[b1, 2 * p + 1, pl.ds(c * 16, 16)] = h1[c] * hi
            pltpu.async_copy(hb.at[b1], agg_sh.at[dv.at[b1]], ss, add=True)

        return 0

    lax.fori_loop(0, NCHUNK + 1, _iter, 0)
    pltpu.make_async_copy(hb.at[0], agg_sh.at[dv.at[0]], ss).wait()
    pltpu.make_async_copy(hb.at[0], agg_sh.at[dv.at[0]], ss).wait()
    plsc.subcore_barrier()

    pltpu.sync_copy(
        agg_sh.at[pl.ds(sid * ROWS_PER_TILE, ROWS_PER_TILE), :],
        out_hbm.at[cid, sid])


_sc_segment = functools.partial(
    pl.kernel,
    out_type=jax.ShapeDtypeStruct((NC, NS, ROWS_PER_TILE, D), jnp.float32),
    mesh=plsc.VectorSubcoreMesh(core_axis_name="c", subcore_axis_name="s"),
    scratch_types=[
        pltpu.VMEM((NEB, CHUNK), jnp.int32),
        pltpu.VMEM((NBUF, CHUNK), jnp.int32),
        pltpu.VMEM((2, CHUNK), jnp.int32),
        pltpu.VMEM((NBUF, CHUNK, D), jnp.float32),
        pltpu.VMEM((NEB, CHUNK // 2, D), jnp.int32),
        pltpu.VMEM((ZROWS, D), jnp.float32),
        pltpu.VMEM_SHARED((N, D), jnp.float32),
        pltpu.SemaphoreType.DMA,
        pltpu.SemaphoreType.DMA,
        pltpu.SemaphoreType.DMA,
        pltpu.SemaphoreType.DMA,
    ],
)(_sc_body)


def kernel(x, edge_attr, edge_index, W1_0, We_0, W2_0, W1_1, We_1, W2_1):
    packed = (edge_index[1] << 14) | edge_index[0]

    ew0 = _ew_packed(edge_attr, We_0)
    h0 = _matmul(x, W1_0, block_rows=1000)
    p0 = _sc_segment(h0, ew0, packed).reshape(NC, N, D)
    ew1 = _ew_packed(edge_attr, We_1)
    x1, h1 = _epilogue_h(p0, W2_0, x, W1_1)
    p1 = _sc_segment(h1, ew1, packed).reshape(NC, N, D)
    return _epilogue(p1, W2_1, x1)


# speedup vs baseline: 1.0179x; 1.0083x over previous
import functools
import math

import jax
import jax.numpy as jnp
from jax import lax
from jax.experimental import pallas as pl
from jax.experimental.pallas import tpu as pltpu
from jax.experimental.pallas import tpu_sc as plsc

N = 10000
E = 320000
D = 128
DE = 16
INV_SQRT_DEG = 1.0 / math.sqrt(32.0)
LOG2 = math.log(2.0)

NC = 2
NS = 16
NW = NC * NS
EPW = E // NW
CHUNK = 80
NCHUNK = EPW // CHUNK
ROWS_PER_TILE = N // NS
ZROWS = 25


def _mm_kernel(x_ref, w_ref, o_ref):
    o_ref[...] = jnp.dot(x_ref[...], w_ref[...],
                         preferred_element_type=jnp.float32)


def _pack_bf16_pairs(z):
    u = lax.bitcast_convert_type(z.astype(jnp.bfloat16),
                                 jnp.uint16).astype(jnp.uint32)
    u3 = u.reshape(u.shape[0] // 2, 2, D)
    return (u3[:, 0, :] | (u3[:, 1, :] << 16)).astype(jnp.int32)


def _ew_kernel(ea_ref, w_ref, o_ref):
    e2 = ea_ref[...]
    ea = e2.reshape(2 * e2.shape[0], DE)
    o_ref[...] = _pack_bf16_pairs(
        jnp.dot(ea, w_ref[...], preferred_element_type=jnp.float32))


def _ew_packed(edge_attr, we, block_rows=8000):
    ea3 = edge_attr.reshape(E // 2, 2, DE)
    return pl.pallas_call(
        _ew_kernel,
        grid=(E // 2 // block_rows,),
        in_specs=[
            pl.BlockSpec((block_rows, 2, DE), lambda i: (i, 0, 0)),
            pl.BlockSpec((DE, D), lambda i: (0, 0)),
        ],
        out_specs=pl.BlockSpec((block_rows, D), lambda i: (i, 0)),
        out_shape=jax.ShapeDtypeStruct((E // 2, D), jnp.int32),
    )(ea3, we)


def _matmul(x, w, block_rows):
    rows, k = x.shape
    kk, cols = w.shape
    return pl.pallas_call(
        _mm_kernel,
        grid=(rows // block_rows,),
        in_specs=[
            pl.BlockSpec((block_rows, k), lambda i: (i, 0)),
            pl.BlockSpec((kk, cols), lambda i: (0, 0)),
        ],
        out_specs=pl.BlockSpec((block_rows, cols), lambda i: (i, 0)),
        out_shape=jax.ShapeDtypeStruct((rows, cols), jnp.float32),
    )(x, w)


def _epilogue_kernel(p_ref, w_ref, x_ref, o_ref):
    a = (p_ref[0] + p_ref[1]) * INV_SQRT_DEG
    z = jnp.dot(a, w_ref[...], preferred_element_type=jnp.float32)
    o_ref[...] = x_ref[...] + jax.nn.softplus(z) - LOG2


def _epilogue(partials, w2, x, block_rows=2000):
    return pl.pallas_call(
        _epilogue_kernel,
        grid=(N // block_rows,),
        in_specs=[
            pl.BlockSpec((NC, block_rows, D), lambda i: (0, i, 0)),
            pl.BlockSpec((D, D), lambda i: (0, 0)),
            pl.BlockSpec((block_rows, D), lambda i: (i, 0)),
        ],
        out_specs=pl.BlockSpec((block_rows, D), lambda i: (i, 0)),
        out_shape=jax.ShapeDtypeStruct((N, D), jnp.float32),
    )(partials, w2, x)


def _epi_h_kernel(p_ref, w_ref, x_ref, w1_ref, x_out, h_out):
    a = (p_ref[0] + p_ref[1]) * INV_SQRT_DEG
    z = jnp.dot(a, w_ref[...], preferred_element_type=jnp.float32)
    xn = x_ref[...] + jax.nn.softplus(z) - LOG2
    x_out[...] = xn
    h_out[...] = jnp.dot(xn, w1_ref[...], preferred_element_type=jnp.float32)


def _epilogue_h(partials, w2, x, w1_next, block_rows=2000):
    return pl.pallas_call(
        _epi_h_kernel,
        grid=(N // block_rows,),
        in_specs=[
            pl.BlockSpec((NC, block_rows, D), lambda i: (0, i, 0)),
            pl.BlockSpec((D, D), lambda i: (0, 0)),
            pl.BlockSpec((block_rows, D), lambda i: (i, 0)),
            pl.BlockSpec((D, D), lambda i: (0, 0)),
        ],
        out_specs=[
            pl.BlockSpec((block_rows, D), lambda i: (i, 0)),
            pl.BlockSpec((block_rows, D), lambda i: (i, 0)),
        ],
        out_shape=[
            jax.ShapeDtypeStruct((N, D), jnp.float32),
            jax.ShapeDtypeStruct((N, D), jnp.float32),
        ],
    )(partials, w2, x, w1_next)


NBUF = 3
NEB = 2


def _sc_body(h_hbm, ew_hbm, pk_hbm, out_hbm,
             sv, dv, pkv, hb, eb, zero_v, agg_sh, sg, se, ss, sp):
    cid = lax.axis_index("c")
    sid = lax.axis_index("s")
    wid = sid * NC + cid

    zline = jnp.zeros((16,), jnp.float32)

    def _zrow(r, _):
        for c in range(D // 16):
            zero_v[r, pl.ds(c * 16, 16)] = zline
        return 0

    lax.fori_loop(0, ZROWS, _zrow, 0)

    def _zfire(b, _):
        pltpu.async_copy(
            zero_v,
            agg_sh.at[pl.ds(sid * ROWS_PER_TILE + b * ZROWS, ZROWS), :], sg)
        return 0

    def _zdrain(b, _):
        pltpu.make_async_copy(
            zero_v, agg_sh.at[pl.ds(sid * ROWS_PER_TILE, ZROWS), :], sg).wait()
        return 0

    lax.fori_loop(0, ROWS_PER_TILE // ZROWS, _zfire, 0)
    lax.fori_loop(0, ROWS_PER_TILE // ZROWS, _zdrain, 0)
    pltpu.async_copy(
        pk_hbm.at[pl.ds(wid * NCHUNK * CHUNK, CHUNK)], pkv.at[0], sp)
    plsc.subcore_barrier()

    def _iter(j, _):
        b = j % NBUF
        be = j % NEB

        @pl.when(j >= NBUF)
        def _():
            pltpu.make_async_copy(hb.at[b], agg_sh.at[dv.at[b]], ss).wait()

        @pl.when(j < NCHUNK)
        def _():
            bp = j % 2
            pltpu.make_async_copy(
                pk_hbm.at[pl.ds(0, CHUNK)], pkv.at[bp], sp).wait()

            @pl.when(j + 1 < NCHUNK)
            def _():
                pltpu.async_copy(
                    pk_hbm.at[pl.ds((wid * NCHUNK + j + 1) * CHUNK, CHUNK)],
                    pkv.at[1 - bp], sp)

            for i in range(CHUNK // 16):
                s = pl.ds(i * 16, 16)
                v = pkv[bp, s]
                sv[be, s] = v & 0x3FFF
                dv[b, s] = v >> 14
            pltpu.async_copy(h_hbm.at[sv.at[be]], hb.at[b], sg)
            pltpu.async_copy(
                ew_hbm.at[pl.ds((wid * NCHUNK + j) * (CHUNK // 2),
                                CHUNK // 2), :],
                eb.at[be], se)

        @pl.when(j >= 1)
        def _():
            b1 = (j - 1) % NBUF
            be1 = (j - 1) % NEB
            pltpu.make_async_copy(h_hbm.at[sv.at[be1]], hb.at[b1], sg).wait()
            pltpu.make_async_copy(
                ew_hbm.at[pl.ds(0, CHUNK // 2), :], eb.at[be1], se).wait()

            @plsc.parallel_loop(0, CHUNK // 2, step=1, unroll=2)
            def _mul(p):
                w = [eb[be1, p, pl.ds(c * 16, 16)] for c in range(D // 16)]
                h0 = [hb[b1, 2 * p, pl.ds(c * 16, 16)]
                      for c in range(D // 16)]
                h1 = [hb[b1, 2 * p + 1, pl.ds(c * 16, 16)]
                      for c in range(D // 16)]
                for c in range(D // 16):
                    lo = lax.bitcast_convert_type(w[c] << 16, jnp.float32)
                    hi = lax.bitcast_convert_type(
                        w[c] & jnp.int32(-65536), jnp.float32)
                    hb[b1, 2 * p, pl.ds(c * 16, 16)] = h0[c] * lo
                    hb[b1, 2 * p + 1, pl.ds(c * 16, 16)] = h1[c] * hi
            pltpu.async_copy(hb.at[b1], agg_sh.at[dv.at[b1]], ss, add=True)

        return 0

    lax.fori_loop(0, NCHUNK + 1, _iter, 0)
    pltpu.make_async_copy(hb.at[0], agg_sh.at[dv.at[0]], ss).wait()
    pltpu.make_async_copy(hb.at[0], agg_sh.at[dv.at[0]], ss).wait()
    plsc.subcore_barrier()

    pltpu.sync_copy(
        agg_sh.at[pl.ds(sid * ROWS_PER_TILE, ROWS_PER_TILE), :],
        out_hbm.at[cid, sid])


_sc_segment = functools.partial(
    pl.kernel,
    out_type=jax.ShapeDtypeStruct((NC, NS, ROWS_PER_TILE, D), jnp.float32),
    mesh=plsc.VectorSubcoreMesh(core_axis_name="c", subcore_axis_name="s"),
    scratch_types=[
        pltpu.VMEM((NEB, CHUNK), jnp.int32),
        pltpu.VMEM((NBUF, CHUNK), jnp.int32),
        pltpu.VMEM((2, CHUNK), jnp.int32),
        pltpu.VMEM((NBUF, CHUNK, D), jnp.float32),
        pltpu.VMEM((NEB, CHUNK // 2, D), jnp.int32),
        pltpu.VMEM((ZROWS, D), jnp.float32),
        pltpu.VMEM_SHARED((N, D), jnp.float32),
        pltpu.SemaphoreType.DMA,
        pltpu.SemaphoreType.DMA,
        pltpu.SemaphoreType.DMA,
        pltpu.SemaphoreType.DMA,
    ],
)(_sc_body)


def kernel(x, edge_attr, edge_index, W1_0, We_0, W2_0, W1_1, We_1, W2_1):
    packed = (edge_index[1] << 14) | edge_index[0]

    ew0 = _ew_packed(edge_attr, We_0)
    h0 = _matmul(x, W1_0, block_rows=2000)
    p0 = _sc_segment(h0, ew0, packed).reshape(NC, N, D)
    ew1 = _ew_packed(edge_attr, We_1)
    x1, h1 = _epilogue_h(p0, W2_0, x, W1_1)
    p1 = _sc_segment(h1, ew1, packed).reshape(NC, N, D)
    return _epilogue(p1, W2_1, x1)
